# baseline (speedup 1.0000x reference)
import functools

import jax
import jax.numpy as jnp
from jax import lax
from jax.experimental import pallas as pl
from jax.experimental.pallas import tpu as pltpu
from jax.experimental.pallas import tpu_sc as plsc

POOL = 16384
N_IN = 65536
NC = 2
NS = 16
NW = NC * NS
L = 16

FEAT_ROWS = 8 * 128
COORD_ROWS = 8 * 3
ROWS_PER_W = FEAT_ROWS // NW


def _sc_gather(feat2d, coord2d, idx):
    mesh = plsc.VectorSubcoreMesh(core_axis_name="c", subcore_axis_name="s")

    @functools.partial(
        pl.kernel,
        out_type=(
            jax.ShapeDtypeStruct((FEAT_ROWS, POOL), jnp.float32),
            jax.ShapeDtypeStruct((COORD_ROWS, POOL), jnp.float32),
        ),
        mesh=mesh,
        scratch_types=[
            pltpu.VMEM((POOL,), jnp.int32),
            pltpu.VMEM((N_IN,), jnp.float32),
            pltpu.VMEM((POOL,), jnp.float32),
        ],
        compiler_params=pltpu.CompilerParams(needs_layout_passes=False),
    )
    def k(feat_hbm, coord_hbm, idx_hbm, outf_hbm, outc_hbm, idx_v, row_v, out_v):
        wid = lax.axis_index("s") * NC + lax.axis_index("c")
        pltpu.sync_copy(idx_hbm, idx_v)

        def gather_row():
            def body(j, carry):
                iv = idx_v[pl.ds(j * L, L)]
                out_v[pl.ds(j * L, L)] = plsc.load_gather(row_v, [iv])
                return carry
            lax.fori_loop(0, POOL // L, body, 0)

        def row_body(r, carry):
            row = wid * ROWS_PER_W + r
            pltpu.sync_copy(feat_hbm.at[row], row_v)
            gather_row()
            pltpu.sync_copy(out_v, outf_hbm.at[row])
            return carry
        lax.fori_loop(0, ROWS_PER_W, row_body, 0)

        @pl.when(wid < COORD_ROWS)
        def _():
            pltpu.sync_copy(coord_hbm.at[wid], row_v)
            gather_row()
            pltpu.sync_copy(out_v, outc_hbm.at[wid])

    return k(feat2d, coord2d, idx)


def kernel(critic_voxel_feature, critic_voxel_coordinate):
    N = critic_voxel_feature.shape[2]
    sample_idx = jax.random.permutation(jax.random.key(42), N)[:POOL]
    sample_idx = sample_idx.astype(jnp.int32)
    feat2d = critic_voxel_feature.reshape(FEAT_ROWS, N)
    coord2d = critic_voxel_coordinate.reshape(COORD_ROWS, N)
    outf, outc = _sc_gather(feat2d, coord2d, sample_idx)
    return (
        outf.reshape(critic_voxel_feature.shape[0], critic_voxel_feature.shape[1], POOL),
        outc.reshape(critic_voxel_coordinate.shape[0], critic_voxel_coordinate.shape[1], POOL),
    )


# speedup vs baseline: 1.5269x; 1.5269x over previous
import functools

import jax
import jax.numpy as jnp
from jax import lax
from jax.experimental import pallas as pl
from jax.experimental.pallas import tpu as pltpu
from jax.experimental.pallas import tpu_sc as plsc

POOL = 16384
N_IN = 65536
NC = 2
NS = 16
NW = NC * NS
L = 16

FEAT_ROWS = 8 * 128
COORD_ROWS = 8 * 3
ROWS_PER_W = FEAT_ROWS // NW


def _sc_gather(feat2d, coord2d, idx):
    mesh = plsc.VectorSubcoreMesh(core_axis_name="c", subcore_axis_name="s")

    @functools.partial(
        pl.kernel,
        out_type=(
            jax.ShapeDtypeStruct((FEAT_ROWS, POOL), jnp.float32),
            jax.ShapeDtypeStruct((COORD_ROWS, POOL), jnp.float32),
        ),
        mesh=mesh,
        scratch_types=[
            pltpu.VMEM((POOL,), jnp.int32),
            pltpu.VMEM((N_IN,), jnp.float32),
            pltpu.VMEM((2, POOL), jnp.float32),
            pltpu.SemaphoreType.DMA,
            pltpu.SemaphoreType.DMA,
            pltpu.SemaphoreType.DMA,
        ],
        compiler_params=pltpu.CompilerParams(needs_layout_passes=False),
    )
    def k(feat_hbm, coord_hbm, idx_hbm, outf_hbm, outc_hbm,
          idx_v, row_v, out_v, sem_in, sem_o0, sem_o1):
        wid = lax.axis_index("s") * NC + lax.axis_index("c")
        pltpu.sync_copy(idx_hbm, idx_v)

        has_coord = wid < COORD_ROWS
        n_rows = jnp.where(has_coord, ROWS_PER_W + 1, ROWS_PER_W)

        def start_in(r):
            @pl.when(r < ROWS_PER_W)
            def _():
                pltpu.async_copy(feat_hbm.at[wid * ROWS_PER_W + r], row_v, sem_in)

            @pl.when(r >= ROWS_PER_W)
            def _():
                pltpu.async_copy(coord_hbm.at[wid], row_v, sem_in)

        def wait_in():
            pltpu.make_async_copy(feat_hbm.at[0], row_v, sem_in).wait()

        def start_out(r, b):
            @pl.when(b == 0)
            def _():
                @pl.when(r < ROWS_PER_W)
                def _():
                    pltpu.async_copy(out_v.at[0], outf_hbm.at[wid * ROWS_PER_W + r], sem_o0)

                @pl.when(r >= ROWS_PER_W)
                def _():
                    pltpu.async_copy(out_v.at[0], outc_hbm.at[wid], sem_o0)

            @pl.when(b == 1)
            def _():
                @pl.when(r < ROWS_PER_W)
                def _():
                    pltpu.async_copy(out_v.at[1], outf_hbm.at[wid * ROWS_PER_W + r], sem_o1)

                @pl.when(r >= ROWS_PER_W)
                def _():
                    pltpu.async_copy(out_v.at[1], outc_hbm.at[wid], sem_o1)

        def wait_out(b):
            @pl.when(b == 0)
            def _():
                pltpu.make_async_copy(out_v.at[0], outf_hbm.at[0], sem_o0).wait()

            @pl.when(b == 1)
            def _():
                pltpu.make_async_copy(out_v.at[1], outf_hbm.at[0], sem_o1).wait()

        def gather_into(b):
            def body(j):
                iv = idx_v[pl.ds(j * L, L)]
                out_v[b, pl.ds(j * L, L)] = plsc.load_gather(row_v, [iv])
            plsc.parallel_loop(0, POOL // L, 1, unroll=8)(body)

        start_in(jnp.int32(0))

        def row_body(r, carry):
            b = lax.rem(r, 2)
            wait_in()

            @pl.when(r >= 2)
            def _():
                wait_out(b)

            gather_into(b)
            start_out(r, b)

            @pl.when(r + 1 < n_rows)
            def _():
                start_in(r + 1)

            return carry

        lax.fori_loop(0, n_rows, row_body, jnp.int32(0))
        wait_out(lax.rem(n_rows, 2))
        wait_out(lax.rem(n_rows + 1, 2))

    return k(feat2d, coord2d, idx)


def kernel(critic_voxel_feature, critic_voxel_coordinate):
    N = critic_voxel_feature.shape[2]
    sample_idx = jax.random.permutation(jax.random.key(42), N)[:POOL]
    sample_idx = sample_idx.astype(jnp.int32)
    feat2d = critic_voxel_feature.reshape(FEAT_ROWS, N)
    coord2d = critic_voxel_coordinate.reshape(COORD_ROWS, N)
    outf, outc = _sc_gather(feat2d, coord2d, sample_idx)
    return (
        outf.reshape(critic_voxel_feature.shape[0], critic_voxel_feature.shape[1], POOL),
        outc.reshape(critic_voxel_coordinate.shape[0], critic_voxel_coordinate.shape[1], POOL),
    )


# speedup vs baseline: 1.9704x; 1.2905x over previous
import functools

import jax
import jax.numpy as jnp
import numpy as np
from jax import lax
from jax.experimental import pallas as pl
from jax.experimental.pallas import tpu as pltpu
from jax.experimental.pallas import tpu_sc as plsc

POOL = 16384
N_IN = 65536
NC = 2
NS = 16
NW = NC * NS
L = 16

FEAT_ROWS = 8 * 128
COORD_ROWS = 8 * 3
ROWS_PER_W = FEAT_ROWS // NW

_SAMPLE_IDX = np.asarray(
    jax.random.permutation(jax.random.key(42), N_IN)[:POOL], dtype=np.int32)


def _sc_gather(feat2d, coord2d, idx):
    mesh = plsc.VectorSubcoreMesh(core_axis_name="c", subcore_axis_name="s")

    @functools.partial(
        pl.kernel,
        out_type=(
            jax.ShapeDtypeStruct((FEAT_ROWS, POOL), jnp.float32),
            jax.ShapeDtypeStruct((COORD_ROWS, POOL), jnp.float32),
        ),
        mesh=mesh,
        scratch_types=[
            pltpu.VMEM((POOL,), jnp.int32),
            pltpu.VMEM((N_IN,), jnp.float32),
            pltpu.VMEM((2, POOL), jnp.float32),
            pltpu.SemaphoreType.DMA,
            pltpu.SemaphoreType.DMA,
            pltpu.SemaphoreType.DMA,
        ],
        compiler_params=pltpu.CompilerParams(needs_layout_passes=False),
    )
    def k(feat_hbm, coord_hbm, idx_hbm, outf_hbm, outc_hbm,
          idx_v, row_v, out_v, sem_in, sem_o0, sem_o1):
        wid = lax.axis_index("s") * NC + lax.axis_index("c")
        pltpu.sync_copy(idx_hbm, idx_v)

        has_coord = wid < COORD_ROWS
        n_rows = jnp.where(has_coord, ROWS_PER_W + 1, ROWS_PER_W)

        def start_in(r):
            @pl.when(r < ROWS_PER_W)
            def _():
                pltpu.async_copy(feat_hbm.at[wid * ROWS_PER_W + r], row_v, sem_in)

            @pl.when(r >= ROWS_PER_W)
            def _():
                pltpu.async_copy(coord_hbm.at[wid], row_v, sem_in)

        def wait_in():
            pltpu.make_async_copy(feat_hbm.at[0], row_v, sem_in).wait()

        def start_out(r, b):
            @pl.when(b == 0)
            def _():
                @pl.when(r < ROWS_PER_W)
                def _():
                    pltpu.async_copy(out_v.at[0], outf_hbm.at[wid * ROWS_PER_W + r], sem_o0)

                @pl.when(r >= ROWS_PER_W)
                def _():
                    pltpu.async_copy(out_v.at[0], outc_hbm.at[wid], sem_o0)

            @pl.when(b == 1)
            def _():
                @pl.when(r < ROWS_PER_W)
                def _():
                    pltpu.async_copy(out_v.at[1], outf_hbm.at[wid * ROWS_PER_W + r], sem_o1)

                @pl.when(r >= ROWS_PER_W)
                def _():
                    pltpu.async_copy(out_v.at[1], outc_hbm.at[wid], sem_o1)

        def wait_out(b):
            @pl.when(b == 0)
            def _():
                pltpu.make_async_copy(out_v.at[0], outf_hbm.at[0], sem_o0).wait()

            @pl.when(b == 1)
            def _():
                pltpu.make_async_copy(out_v.at[1], outf_hbm.at[0], sem_o1).wait()

        def gather_into(b):
            def body(j):
                iv = idx_v[pl.ds(j * L, L)]
                out_v[b, pl.ds(j * L, L)] = plsc.load_gather(row_v, [iv])
            plsc.parallel_loop(0, POOL // L, 1, unroll=8)(body)

        start_in(jnp.int32(0))

        def row_body(r, carry):
            b = lax.rem(r, 2)
            wait_in()

            @pl.when(r >= 2)
            def _():
                wait_out(b)

            gather_into(b)
            start_out(r, b)

            @pl.when(r + 1 < n_rows)
            def _():
                start_in(r + 1)

            return carry

        lax.fori_loop(0, n_rows, row_body, jnp.int32(0))
        wait_out(lax.rem(n_rows, 2))
        wait_out(lax.rem(n_rows + 1, 2))

    return k(feat2d, coord2d, idx)


def kernel(critic_voxel_feature, critic_voxel_coordinate):
    N = critic_voxel_feature.shape[2]
    sample_idx = jnp.asarray(_SAMPLE_IDX)
    feat2d = critic_voxel_feature.reshape(FEAT_ROWS, N)
    coord2d = critic_voxel_coordinate.reshape(COORD_ROWS, N)
    outf, outc = _sc_gather(feat2d, coord2d, sample_idx)
    return (
        outf.reshape(critic_voxel_feature.shape[0], critic_voxel_feature.shape[1], POOL),
        outc.reshape(critic_voxel_coordinate.shape[0], critic_voxel_coordinate.shape[1], POOL),
    )


# speedup vs baseline: 2.1746x; 1.1036x over previous
import functools

import jax
import jax.numpy as jnp
import numpy as np
from jax import lax
from jax.experimental import pallas as pl
from jax.experimental.pallas import tpu as pltpu
from jax.experimental.pallas import tpu_sc as plsc

POOL = 16384
N_IN = 65536
NC = 2
NS = 16
NW = NC * NS
L = 16

FEAT_ROWS = 8 * 128
COORD_ROWS = 8 * 3
ROWS_PER_W = FEAT_ROWS // NW

_SAMPLE_IDX = np.asarray(
    jax.random.permutation(jax.random.key(42), N_IN)[:POOL], dtype=np.int32)


def _sc_gather(feat2d, coord2d, idx):
    mesh = plsc.VectorSubcoreMesh(core_axis_name="c", subcore_axis_name="s")

    @functools.partial(
        pl.kernel,
        out_type=(
            jax.ShapeDtypeStruct((8, FEAT_ROWS // 8, POOL), jnp.float32),
            jax.ShapeDtypeStruct((8, COORD_ROWS // 8, POOL), jnp.float32),
        ),
        mesh=mesh,
        scratch_types=[
            pltpu.VMEM((POOL,), jnp.int32),
            pltpu.VMEM((N_IN,), jnp.float32),
            pltpu.VMEM((2, POOL), jnp.float32),
            pltpu.SemaphoreType.DMA,
            pltpu.SemaphoreType.DMA,
            pltpu.SemaphoreType.DMA,
        ],
        compiler_params=pltpu.CompilerParams(needs_layout_passes=False),
    )
    def k(feat_hbm, coord_hbm, idx_hbm, outf_hbm, outc_hbm,
          idx_v, row_v, out_v, sem_in, sem_o0, sem_o1):
        wid = lax.axis_index("s") * NC + lax.axis_index("c")
        pltpu.sync_copy(idx_hbm, idx_v)

        has_coord = wid < COORD_ROWS
        n_rows = jnp.where(has_coord, ROWS_PER_W + 1, ROWS_PER_W)

        QTR = N_IN // 4

        def start_in(r):
            @pl.when(r < ROWS_PER_W)
            def _():
                row = wid * ROWS_PER_W + r
                a, b = lax.div(row, FEAT_ROWS // 8), lax.rem(row, FEAT_ROWS // 8)
                for q in range(4):
                    pltpu.async_copy(feat_hbm.at[a, b, pl.ds(q * QTR, QTR)],
                                     row_v.at[pl.ds(q * QTR, QTR)], sem_in)

            @pl.when(r >= ROWS_PER_W)
            def _():
                a, b = lax.div(wid, COORD_ROWS // 8), lax.rem(wid, COORD_ROWS // 8)
                for q in range(4):
                    pltpu.async_copy(coord_hbm.at[a, b, pl.ds(q * QTR, QTR)],
                                     row_v.at[pl.ds(q * QTR, QTR)], sem_in)

        def wait_in():
            for _ in range(4):
                pltpu.make_async_copy(feat_hbm.at[0, 0, pl.ds(0, QTR)],
                                      row_v.at[pl.ds(0, QTR)], sem_in).wait()

        def start_out(r, b):
            @pl.when(b == 0)
            def _():
                @pl.when(r < ROWS_PER_W)
                def _():
                    row = wid * ROWS_PER_W + r
                    pltpu.async_copy(out_v.at[0], outf_hbm.at[lax.div(row, FEAT_ROWS // 8), lax.rem(row, FEAT_ROWS // 8)], sem_o0)

                @pl.when(r >= ROWS_PER_W)
                def _():
                    pltpu.async_copy(out_v.at[0], outc_hbm.at[lax.div(wid, COORD_ROWS // 8), lax.rem(wid, COORD_ROWS // 8)], sem_o0)

            @pl.when(b == 1)
            def _():
                @pl.when(r < ROWS_PER_W)
                def _():
                    row = wid * ROWS_PER_W + r
                    pltpu.async_copy(out_v.at[1], outf_hbm.at[lax.div(row, FEAT_ROWS // 8), lax.rem(row, FEAT_ROWS // 8)], sem_o1)

                @pl.when(r >= ROWS_PER_W)
                def _():
                    pltpu.async_copy(out_v.at[1], outc_hbm.at[lax.div(wid, COORD_ROWS // 8), lax.rem(wid, COORD_ROWS // 8)], sem_o1)

        def wait_out(b):
            @pl.when(b == 0)
            def _():
                pltpu.make_async_copy(out_v.at[0], outf_hbm.at[0, 0], sem_o0).wait()

            @pl.when(b == 1)
            def _():
                pltpu.make_async_copy(out_v.at[1], outf_hbm.at[0, 0], sem_o1).wait()

        def gather_into(b):
            def body(j):
                iv = idx_v[pl.ds(j * L, L)]
                out_v[b, pl.ds(j * L, L)] = plsc.load_gather(row_v, [iv])
            plsc.parallel_loop(0, POOL // L, 1, unroll=8)(body)

        start_in(jnp.int32(0))

        def row_body(r, carry):
            b = lax.rem(r, 2)
            wait_in()

            @pl.when(r >= 2)
            def _():
                wait_out(b)

            gather_into(b)
            start_out(r, b)

            @pl.when(r + 1 < n_rows)
            def _():
                start_in(r + 1)

            return carry

        lax.fori_loop(0, n_rows, row_body, jnp.int32(0))
        wait_out(lax.rem(n_rows, 2))
        wait_out(lax.rem(n_rows + 1, 2))

    return k(feat2d, coord2d, idx)


def kernel(critic_voxel_feature, critic_voxel_coordinate):
    sample_idx = jnp.asarray(_SAMPLE_IDX)
    outf, outc = _sc_gather(critic_voxel_feature, critic_voxel_coordinate, sample_idx)
    return (outf, outc)
